# baseline (speedup 1.0000x reference)
import functools

import jax
import jax.numpy as jnp
from jax import lax
from jax.experimental import pallas as pl
from jax.experimental.pallas import tpu as pltpu
from jax.experimental.pallas import tpu_sc as plsc

NUM_WORKERS = 32
LANES = 16
CHUNK = 104


def _pooled_sum_kernel(batch, embed_dim, num_tables):
    ex_per_w = batch // NUM_WORKERS
    n_lane_groups = embed_dim // LANES

    mesh = plsc.VectorSubcoreMesh(core_axis_name="c", subcore_axis_name="s")

    @functools.partial(
        pl.kernel,
        out_type=jax.ShapeDtypeStruct((batch, embed_dim), jnp.float32),
        mesh=mesh,
        scratch_types=[
            pltpu.VMEM((2 * ex_per_w, CHUNK), jnp.int32),
            pltpu.VMEM((CHUNK, embed_dim), jnp.float32),
            pltpu.VMEM((ex_per_w, embed_dim), jnp.float32),
            pltpu.SemaphoreType.DMA,
        ],
        compiler_params=pltpu.CompilerParams(use_tc_tiling_on_sc=False),
    )
    def body(ids_hbm, table_hbm, out_hbm, idx_v, rows_v, acc_v, sem):
        wid = lax.axis_index("s") * 2 + lax.axis_index("c")
        pltpu.sync_copy(ids_hbm.at[wid], idx_v)

        def ebody(e, carry):
            zero = jnp.zeros((LANES,), jnp.float32)
            s = (zero,) * n_lane_groups
            for h, nreal in ((0, CHUNK), (1, 200 - CHUNK)):
                j = 2 * e + h
                pltpu.async_copy(table_hbm.at[idx_v.at[j]], rows_v, sem).wait()

                def tbody(t, c):
                    return tuple(
                        c[l] + rows_v[t, pl.ds(LANES * l, LANES)]
                        for l in range(n_lane_groups)
                    )

                s = lax.fori_loop(0, nreal, tbody, s)
            for l in range(n_lane_groups):
                acc_v[e, pl.ds(LANES * l, LANES)] = s[l]
            return carry

        lax.fori_loop(0, ex_per_w, ebody, 0)
        pltpu.sync_copy(acc_v, out_hbm.at[pl.ds(wid * ex_per_w, ex_per_w)])

    return body


def _classifier_kernel(x_ref, w_ref, b_ref, o_ref):
    acc = jnp.dot(x_ref[...], w_ref[...], preferred_element_type=jnp.float32)
    o_ref[...] = acc * (1.0 / 200.0) + b_ref[...]


def kernel(subword_ids, table, W, b):
    batch, seq_len = subword_ids.shape
    num_tables, embed_dim = table.shape
    num_classes = W.shape[1]

    ids_pad = jnp.pad(subword_ids, ((0, 0), (0, 2 * CHUNK - seq_len)))
    ids3 = ids_pad.reshape(NUM_WORKERS, (batch // NUM_WORKERS) * 2, CHUNK)

    pooled = _pooled_sum_kernel(batch, embed_dim, num_tables)(ids3, table)

    logits = pl.pallas_call(
        _classifier_kernel,
        out_shape=jax.ShapeDtypeStruct((batch, num_classes), jnp.float32),
    )(pooled, W, b.reshape(1, num_classes))
    return logits


# speedup vs baseline: 1.0040x; 1.0040x over previous
import functools

import jax
import jax.numpy as jnp
from jax import lax
from jax.experimental import pallas as pl
from jax.experimental.pallas import tpu as pltpu
from jax.experimental.pallas import tpu_sc as plsc

NUM_WORKERS = 32
LANES = 16
CHUNK = 104
UNROLL = 8


def _pooled_sum_kernel(batch, embed_dim, seq_len):
    ex_per_w = batch // NUM_WORKERS
    n_lg = embed_dim // LANES
    n_real1 = seq_len - CHUNK

    mesh = plsc.VectorSubcoreMesh(core_axis_name="c", subcore_axis_name="s")

    @functools.partial(
        pl.kernel,
        out_type=jax.ShapeDtypeStruct((batch, embed_dim), jnp.float32),
        mesh=mesh,
        scratch_types=[
            pltpu.VMEM((2 * ex_per_w, CHUNK), jnp.int32),
            pltpu.VMEM((CHUNK, embed_dim), jnp.float32),
            pltpu.VMEM((CHUNK, embed_dim), jnp.float32),
            pltpu.VMEM((ex_per_w, embed_dim), jnp.float32),
            pltpu.SemaphoreType.DMA,
            pltpu.SemaphoreType.DMA,
        ],
        compiler_params=pltpu.CompilerParams(use_tc_tiling_on_sc=False),
    )
    def body(ids_hbm, table_hbm, out_hbm, idx_v, rows_a, rows_b, acc_v,
             sem_a, sem_b):
        wid = lax.axis_index("s") * 2 + lax.axis_index("c")
        pltpu.sync_copy(ids_hbm.at[wid], idx_v)

        def reduce_rows(rows, ntok, s):
            def tbody(t, c):
                base = t * UNROLL
                for k in range(UNROLL):
                    c = tuple(
                        c[l] + rows[base + k, pl.ds(LANES * l, LANES)]
                        for l in range(n_lg)
                    )
                return c

            return lax.fori_loop(0, ntok // UNROLL, tbody, s)

        pltpu.async_copy(table_hbm.at[idx_v.at[0]], rows_a, sem_a)

        zero = jnp.zeros((LANES,), jnp.float32)

        def ebody(e, carry):
            pltpu.async_copy(table_hbm.at[idx_v.at[2 * e + 1]], rows_b, sem_b)
            pltpu.make_async_copy(table_hbm.at[idx_v.at[2 * e]], rows_a,
                                  sem_a).wait()
            s = reduce_rows(rows_a, CHUNK, (zero,) * n_lg)

            @pl.when(e < ex_per_w - 1)
            def _():
                pltpu.async_copy(table_hbm.at[idx_v.at[2 * e + 2]], rows_a,
                                 sem_a)

            pltpu.make_async_copy(table_hbm.at[idx_v.at[2 * e + 1]], rows_b,
                                  sem_b).wait()
            s = reduce_rows(rows_b, n_real1, s)
            for l in range(n_lg):
                acc_v[e, pl.ds(LANES * l, LANES)] = s[l]
            return carry

        lax.fori_loop(0, ex_per_w, ebody, 0)
        pltpu.sync_copy(acc_v, out_hbm.at[pl.ds(wid * ex_per_w, ex_per_w)])

    return body


def _classifier_kernel(x_ref, w_ref, b_ref, o_ref):
    acc = jnp.dot(x_ref[...], w_ref[...], preferred_element_type=jnp.float32)
    o_ref[...] = acc * (1.0 / 200.0) + b_ref[...]


def kernel(subword_ids, table, W, b):
    batch, seq_len = subword_ids.shape
    embed_dim = table.shape[1]
    num_classes = W.shape[1]

    ids_pad = jnp.pad(subword_ids, ((0, 0), (0, 2 * CHUNK - seq_len)))
    ids3 = ids_pad.reshape(NUM_WORKERS, (batch // NUM_WORKERS) * 2, CHUNK)

    pooled = _pooled_sum_kernel(batch, embed_dim, seq_len)(ids3, table)

    logits = pl.pallas_call(
        _classifier_kernel,
        out_shape=jax.ShapeDtypeStruct((batch, num_classes), jnp.float32),
    )(pooled, W, b.reshape(1, num_classes))
    return logits


# speedup vs baseline: 1.9221x; 1.9144x over previous
import functools

import jax
import jax.numpy as jnp
from jax import lax
from jax.experimental import pallas as pl
from jax.experimental.pallas import tpu as pltpu
from jax.experimental.pallas import tpu_sc as plsc

NUM_WORKERS = 32
LANES = 16
CHUNK = 104
OFF1 = 96
UNROLL = 8
NBUF = 8


def _pooled_sum_kernel(batch, embed_dim, seq_len):
    ex_per_w = batch // NUM_WORKERS
    n_lg = embed_dim // LANES
    n_chunks = 2 * ex_per_w
    n_real1 = seq_len - CHUNK

    mesh = plsc.VectorSubcoreMesh(core_axis_name="c", subcore_axis_name="s")

    @functools.partial(
        pl.kernel,
        out_type=jax.ShapeDtypeStruct((batch, embed_dim), jnp.float32),
        mesh=mesh,
        scratch_types=[
            pltpu.VMEM((ex_per_w, seq_len), jnp.int32),
            [pltpu.VMEM((CHUNK, embed_dim), jnp.float32) for _ in range(NBUF)],
            pltpu.VMEM((ex_per_w, embed_dim), jnp.float32),
            [pltpu.SemaphoreType.DMA for _ in range(NBUF)],
        ],
        compiler_params=pltpu.CompilerParams(use_tc_tiling_on_sc=False),
    )
    def body(ids_hbm, table_hbm, out_hbm, idx_v, rows, acc_v, sems):
        wid = lax.axis_index("s") * 2 + lax.axis_index("c")
        base = wid * ex_per_w
        pltpu.sync_copy(ids_hbm.at[pl.ds(base, ex_per_w)], idx_v)

        def start(e, parity, b):
            off = OFF1 if parity else 0
            pltpu.async_copy(
                table_hbm.at[idx_v.at[e, pl.ds(off, CHUNK)]], rows[b], sems[b]
            )

        def reduce_rows(r, t0, ntok, s):
            def tbody(t, c):
                tb = t0 + t * UNROLL
                for k in range(UNROLL):
                    c = tuple(
                        c[l] + r[tb + k, pl.ds(LANES * l, LANES)]
                        for l in range(n_lg)
                    )
                return c

            return lax.fori_loop(0, ntok // UNROLL, tbody, s)

        for b in range(NBUF):
            start(b // 2, b % 2, b)

        zero = jnp.zeros((LANES,), jnp.float32)

        def gbody(g, carry):
            q0 = g * NBUF
            s = (zero,) * n_lg
            for b in range(NBUF):
                q = q0 + b
                pltpu.make_async_copy(
                    table_hbm.at[pl.ds(0, CHUNK)], rows[b], sems[b]
                ).wait()
                if b % 2 == 0:
                    s = reduce_rows(rows[b], 0, CHUNK, (zero,) * n_lg)
                else:
                    s = reduce_rows(rows[b], CHUNK - n_real1, n_real1, s)
                    e = q // 2
                    for l in range(n_lg):
                        acc_v[e, pl.ds(LANES * l, LANES)] = s[l]

                @pl.when(q + NBUF < n_chunks)
                def _():
                    start((q + NBUF) // 2, b % 2, b)

            return carry

        lax.fori_loop(0, n_chunks // NBUF, gbody, 0)
        pltpu.sync_copy(acc_v, out_hbm.at[pl.ds(base, ex_per_w)])

    return body


def _classifier_kernel(x_ref, w_ref, b_ref, o_ref):
    acc = jnp.dot(x_ref[...], w_ref[...], preferred_element_type=jnp.float32)
    o_ref[...] = acc * (1.0 / 200.0) + b_ref[...]


def kernel(subword_ids, table, W, b):
    batch, seq_len = subword_ids.shape
    embed_dim = table.shape[1]
    num_classes = W.shape[1]

    pooled = _pooled_sum_kernel(batch, embed_dim, seq_len)(subword_ids, table)

    logits = pl.pallas_call(
        _classifier_kernel,
        out_shape=jax.ShapeDtypeStruct((batch, num_classes), jnp.float32),
    )(pooled, W, b.reshape(1, num_classes))
    return logits


# speedup vs baseline: 1.9398x; 1.0092x over previous
import functools

import jax
import jax.numpy as jnp
from jax import lax
from jax.experimental import pallas as pl
from jax.experimental.pallas import tpu as pltpu
from jax.experimental.pallas import tpu_sc as plsc

NUM_WORKERS = 32
LANES = 16
UNROLL = 8
NBUF = 4


def _pooled_sum_kernel(batch, embed_dim, seq_len):
    ex_per_w = batch // NUM_WORKERS
    n_lg = embed_dim // LANES

    mesh = plsc.VectorSubcoreMesh(core_axis_name="c", subcore_axis_name="s")

    @functools.partial(
        pl.kernel,
        out_type=jax.ShapeDtypeStruct((batch, embed_dim), jnp.float32),
        mesh=mesh,
        scratch_types=[
            pltpu.VMEM((ex_per_w, seq_len), jnp.int32),
            [pltpu.VMEM((seq_len, embed_dim), jnp.float32)
             for _ in range(NBUF)],
            pltpu.VMEM((ex_per_w, embed_dim), jnp.float32),
            [pltpu.SemaphoreType.DMA for _ in range(NBUF)],
        ],
        compiler_params=pltpu.CompilerParams(use_tc_tiling_on_sc=False),
    )
    def body(ids_hbm, table_hbm, out_hbm, idx_v, rows, acc_v, sems):
        wid = lax.axis_index("s") * 2 + lax.axis_index("c")
        base = wid * ex_per_w
        pltpu.sync_copy(ids_hbm.at[pl.ds(base, ex_per_w)], idx_v)

        def start(e, b):
            pltpu.async_copy(table_hbm.at[idx_v.at[e]], rows[b], sems[b])

        def reduce_rows(r):
            def tbody(t, c):
                tb = t * UNROLL
                for k in range(UNROLL):
                    c = tuple(
                        c[l] + r[tb + k, pl.ds(LANES * l, LANES)]
                        for l in range(n_lg)
                    )
                return c

            zero = jnp.zeros((LANES,), jnp.float32)
            return lax.fori_loop(0, seq_len // UNROLL, tbody, (zero,) * n_lg)

        for b in range(NBUF):
            start(b, b)

        def gbody(g, carry):
            e0 = g * NBUF
            for b in range(NBUF):
                e = e0 + b
                pltpu.make_async_copy(
                    table_hbm.at[pl.ds(0, seq_len)], rows[b], sems[b]
                ).wait()
                s = reduce_rows(rows[b])
                for l in range(n_lg):
                    acc_v[e, pl.ds(LANES * l, LANES)] = s[l]

                @pl.when(e + NBUF < ex_per_w)
                def _():
                    start(e + NBUF, b)

            return carry

        lax.fori_loop(0, ex_per_w // NBUF, gbody, 0)
        pltpu.sync_copy(acc_v, out_hbm.at[pl.ds(base, ex_per_w)])

    return body


def _classifier_kernel(x_ref, w_ref, b_ref, o_ref):
    acc = jnp.dot(x_ref[...], w_ref[...], preferred_element_type=jnp.float32)
    o_ref[...] = acc * (1.0 / 200.0) + b_ref[...]


def kernel(subword_ids, table, W, b):
    batch, seq_len = subword_ids.shape
    embed_dim = table.shape[1]
    num_classes = W.shape[1]

    pooled = _pooled_sum_kernel(batch, embed_dim, seq_len)(subword_ids, table)

    logits = pl.pallas_call(
        _classifier_kernel,
        out_shape=jax.ShapeDtypeStruct((batch, num_classes), jnp.float32),
    )(pooled, W, b.reshape(1, num_classes))
    return logits
